# baseline (speedup 1.0000x reference)
import functools

import jax
import jax.numpy as jnp
from jax import lax
from jax.experimental import pallas as pl
from jax.experimental.pallas import tpu as pltpu
from jax.experimental.pallas import tpu_sc as plsc

B, T, C, H, W = 32, 256, 3, 32, 32
FEAT = 512
HID = 128
TOP_K = 8
CHW = C * H * W

_DN_T = (((1,), (1,)), ((), ()))


def _dot_t(a, b):
    return jax.lax.dot_general(a, b, _DN_T,
                               preferred_element_type=jnp.float32)


_DN_LT = (((0,), (1,)), ((), ()))

B_BLK = 2


def _enc_kernel(x_ref, wenc_ref, benc_ref, wf_ref, wb_ref, bf_ref, bb_ref,
                out_ref):
    for j in range(B_BLK):
        feats = jax.lax.dot_general(x_ref[j], wenc_ref[...], _DN_LT,
                                    preferred_element_type=jnp.float32)
        feats = feats + benc_ref[...]
        out_ref[j, :, :3 * HID] = _dot_t(feats, wf_ref[...]) + bf_ref[...]
        out_ref[j, :, 3 * HID:] = _dot_t(feats, wb_ref[...]) + bb_ref[...]


def _encode(xt, wenc, benc, wf, wb, bf, bb):
    return pl.pallas_call(
        _enc_kernel,
        grid=(B // B_BLK,),
        in_specs=[
            pl.BlockSpec((B_BLK, CHW, T), lambda k: (k, 0, 0)),
            pl.BlockSpec((FEAT, CHW), lambda k: (0, 0)),
            pl.BlockSpec((1, FEAT), lambda k: (0, 0)),
            pl.BlockSpec((3 * HID, FEAT), lambda k: (0, 0)),
            pl.BlockSpec((3 * HID, FEAT), lambda k: (0, 0)),
            pl.BlockSpec((1, 3 * HID), lambda k: (0, 0)),
            pl.BlockSpec((1, 3 * HID), lambda k: (0, 0)),
        ],
        out_specs=pl.BlockSpec((B_BLK, T, 6 * HID), lambda k: (k, 0, 0)),
        out_shape=jax.ShapeDtypeStruct((B, T, 6 * HID), jnp.float32),
    )(xt, wenc, benc, wf, wb, bf, bb)


SEQ_BLK = 32


def _gru_kernel(gf_ref, gb_ref, whf_ref, whb_ref, bhf_ref, bhb_ref, len_ref,
                outf_ref, outb_ref, h_ref):
    k = pl.program_id(0)

    @pl.when(k == 0)
    def _():
        h_ref[...] = jnp.zeros_like(h_ref)

    lengths = len_ref[...]

    def step(i, _):
        h = h_ref[...]
        gi_f = gf_ref[:, i, :]
        gi_b = gb_ref[:, SEQ_BLK - 1 - i, :]
        gh_f = _dot_t(h[:, :HID], whf_ref[...]) + bhf_ref[...]
        gh_b = _dot_t(h[:, HID:], whb_ref[...]) + bhb_ref[...]

        t_f = k * SEQ_BLK + i
        t_b = T - 1 - t_f

        def gru_dir(gi, gh_d, h_d, t):
            r = jax.nn.sigmoid(gi[:, :HID] + gh_d[:, :HID])
            z = jax.nn.sigmoid(gi[:, HID:2 * HID] + gh_d[:, HID:2 * HID])
            n = jnp.tanh(gi[:, 2 * HID:] + r * gh_d[:, 2 * HID:])
            h_new = (1.0 - z) * n + z * h_d
            valid = lengths > t
            return jnp.where(valid, h_new, h_d)

        h_f = gru_dir(gi_f, gh_f, h[:, :HID], t_f)
        h_b = gru_dir(gi_b, gh_b, h[:, HID:], t_b)
        outf_ref[:, i, :] = h_f
        outb_ref[:, SEQ_BLK - 1 - i, :] = h_b
        h_ref[...] = jnp.concatenate([h_f, h_b], axis=1)
        return 0

    jax.lax.fori_loop(0, SEQ_BLK, step, 0, unroll=True)


def _gru(gcat3, whf, whb, bhf, bhb, lengths_col):
    nb = T // SEQ_BLK
    out = pl.pallas_call(
        _gru_kernel,
        grid=(nb,),
        in_specs=[
            pl.BlockSpec((B, SEQ_BLK, 3 * HID), lambda k: (0, k, 0)),
            pl.BlockSpec((B, SEQ_BLK, 3 * HID), lambda k, nb=nb: (0, nb - 1 - k, 1)),
            pl.BlockSpec((3 * HID, HID), lambda k: (0, 0)),
            pl.BlockSpec((3 * HID, HID), lambda k: (0, 0)),
            pl.BlockSpec((1, 3 * HID), lambda k: (0, 0)),
            pl.BlockSpec((1, 3 * HID), lambda k: (0, 0)),
            pl.BlockSpec((B, 1), lambda k: (0, 0)),
        ],
        out_specs=[
            pl.BlockSpec((B, SEQ_BLK, HID), lambda k: (0, k, 0)),
            pl.BlockSpec((B, SEQ_BLK, HID), lambda k, nb=nb: (0, nb - 1 - k, 0)),
        ],
        out_shape=[
            jax.ShapeDtypeStruct((B, T, HID), jnp.float32),
            jax.ShapeDtypeStruct((B, T, HID), jnp.float32),
        ],
        scratch_shapes=[pltpu.VMEM((B, 2 * HID), jnp.float32)],
        compiler_params=pltpu.CompilerParams(
            dimension_semantics=("arbitrary",)),
    )(gcat3, gcat3, whf, whb, bhf, bhb, lengths_col)
    return out


def _score_kernel(xf_ref, xb_ref, w1_ref, b1_ref, w2_ref, len_ref, temp_ref,
                  probs_ref, lsp_ref):
    xf = xf_ref[...]
    xb = xb_ref[...]
    w1 = w1_ref[...]
    h1 = _dot_t(xf.reshape(B * T, HID), w1[:, :HID])
    h1 = h1 + _dot_t(xb.reshape(B * T, HID), w1[:, HID:])
    h1 = jax.nn.relu(h1 + b1_ref[...])
    scores = jnp.sum(h1.reshape(B, T, 64) * w2_ref[...], axis=2)

    lengths = len_ref[...]
    tpos = jax.lax.broadcasted_iota(jnp.int32, (B, T), 1)
    mask = tpos < lengths
    temp = jnp.clip(temp_ref[0, 0], 0.001, 10.0)
    logits = jnp.where(mask, scores * (1.0 / temp), -jnp.inf)

    m = jnp.max(logits, axis=1, keepdims=True)
    e = jnp.exp(logits - m)
    probs_ref[...] = e / jnp.sum(e, axis=1, keepdims=True)
    lsp_ref[...] = jnp.broadcast_to(lengths, (B, 16))


def _scores(out_f, out_b, w1, b1, w2, lengths_col, temp):
    full = lambda s: pl.BlockSpec(s, lambda *a: tuple(0 for _ in s))
    return pl.pallas_call(
        _score_kernel,
        in_specs=[
            full((B, T, HID)),
            full((B, T, HID)),
            full((64, 2 * HID)),
            full((1, 64)),
            full((1, 1, 64)),
            full((B, 1)),
            pl.BlockSpec(memory_space=pltpu.SMEM),
        ],
        out_specs=[full((B, T)), full((B, 16))],
        out_shape=[jax.ShapeDtypeStruct((B, T), jnp.float32),
                   jax.ShapeDtypeStruct((B, 16), jnp.int32)],
    )(out_f, out_b, w1, b1, w2, lengths_col, temp)


_SC_L = 16
_NCH = T // _SC_L


def _lane_gather(v, idx):
    dn = lax.GatherDimensionNumbers(offset_dims=(),
                                    collapsed_slice_dims=(0,),
                                    start_index_map=(0,))
    return lax.gather(v, idx[:, None], dn, (1,),
                      mode=lax.GatherScatterMode.PROMISE_IN_BOUNDS)


def _splat_reduce(v, op, iota):
    for s in (1, 2, 4, 8):
        v = op(v, _lane_gather(v, iota ^ s))
    return v


def _att_sc(probs, lensplat):
    info = plsc.get_sparse_core_info()
    nc = info.num_cores

    @functools.partial(
        pl.kernel,
        mesh=plsc.VectorSubcoreMesh(core_axis_name="c", subcore_axis_name="s"),
        out_type=jax.ShapeDtypeStruct((B, T), jnp.float32),
        scratch_types=[
            pltpu.VMEM((T,), jnp.float32),
            pltpu.VMEM((T,), jnp.float32),
            pltpu.VMEM((T,), jnp.float32),
            pltpu.VMEM((_SC_L,), jnp.int32),
        ],
    )
    def att_kernel(probs_hbm, lsp_hbm, att_hbm, work_v, orig_v, out_v, len_v):
        wid = lax.axis_index("s") * nc + lax.axis_index("c")
        pltpu.sync_copy(probs_hbm.at[wid], work_v)
        pltpu.sync_copy(probs_hbm.at[wid], orig_v)
        pltpu.sync_copy(lsp_hbm.at[wid], len_v)

        iota = lax.iota(jnp.int32, _SC_L)
        lvalv = len_v[...]
        lfv = lvalv.astype(jnp.float32)

        vsumv = jnp.zeros((_SC_L,), jnp.float32)
        for _ in range(TOP_K):
            chunks = [work_v[pl.ds(c * _SC_L, _SC_L)] for c in range(_NCH)]
            m = chunks[0]
            for c in range(1, _NCH):
                m = jnp.maximum(m, chunks[c])
            gv = _splat_reduce(m, jnp.maximum, iota)
            cmin = jnp.full((_SC_L,), T, jnp.int32)
            for c in range(_NCH):
                cand = jnp.where(chunks[c] == gv, iota + c * _SC_L, T)
                cmin = jnp.minimum(cmin, cand)
            gidxv = _splat_reduce(cmin, jnp.minimum, iota)
            vsumv = vsumv + gv
            for c in range(_NCH):
                upd = jnp.where(iota + c * _SC_L == gidxv,
                                jnp.float32(-1.0), chunks[c])
                work_v[pl.ds(c * _SC_L, _SC_L)] = upd

        inv_vsumv = 1.0 / jnp.maximum(vsumv, jnp.float32(1e-12))
        uvalv = 1.0 / (lfv + jnp.float32(1e-08))
        utv = jnp.where(vsumv > jnp.float32(1e-08),
                        jnp.full((_SC_L,), 1.0, jnp.float32),
                        jnp.full((_SC_L,), 0.0, jnp.float32))
        for c in range(_NCH):
            w = work_v[pl.ds(c * _SC_L, _SC_L)]
            o = orig_v[pl.ds(c * _SC_L, _SC_L)]
            topk_att = jnp.where(w == jnp.full((_SC_L,), -1.0, jnp.float32),
                                 o * inv_vsumv, jnp.float32(0.0))
            uni_att = jnp.where(iota + c * _SC_L < lvalv, uvalv,
                                jnp.float32(0.0))
            out_v[pl.ds(c * _SC_L, _SC_L)] = (topk_att * utv
                                              + uni_att * (1.0 - utv))
        pltpu.sync_copy(out_v, att_hbm.at[wid])

    return att_kernel(probs, lensplat)


def _pool_kernel(xf_ref, xb_ref, att_ref, wt_ref, wo_ref, bt_ref, bo_ref,
                 outt_ref, outo_ref):
    att = att_ref[...]
    seq_f = jnp.sum(xf_ref[...] * att[:, :, None], axis=1)
    seq_b = jnp.sum(xb_ref[...] * att[:, :, None], axis=1)
    wt = wt_ref[...]
    wo = wo_ref[...]
    outt_ref[...] = (_dot_t(seq_f, wt[:, :HID]) + _dot_t(seq_b, wt[:, HID:])
                     + bt_ref[...])
    outo_ref[...] = (_dot_t(seq_f, wo[:, :HID]) + _dot_t(seq_b, wo[:, HID:])
                     + bo_ref[...])


def _pool(out_f, out_b, att, wt, wo, bt, bo):
    full = lambda s: pl.BlockSpec(s, lambda *a: tuple(0 for _ in s))
    return pl.pallas_call(
        _pool_kernel,
        in_specs=[
            full((B, T, HID)),
            full((B, T, HID)),
            full((B, T)),
            full((11, 2 * HID)),
            full((10, 2 * HID)),
            full((1, 11)),
            full((1, 10)),
        ],
        out_specs=[
            full((B, 11)),
            full((B, 10)),
        ],
        out_shape=[
            jax.ShapeDtypeStruct((B, 11), jnp.float32),
            jax.ShapeDtypeStruct((B, 10), jnp.float32),
        ],
    )(out_f, out_b, att, wt, wo, bt, bo)


def kernel(frames, params, lengths):
    xt = jnp.transpose(frames, (0, 2, 3, 4, 1)).reshape(B, CHW, T)
    pf, pb = params['gru_fwd'], params['gru_bwd']

    gcat3 = _encode(xt, params['W_enc'], params['b_enc'].reshape(1, FEAT),
                    pf['W_ih'], pb['W_ih'],
                    pf['b_ih'].reshape(1, 3 * HID),
                    pb['b_ih'].reshape(1, 3 * HID))

    lengths_col = lengths.reshape(B, 1)
    out_f, out_b = _gru(gcat3, pf['W_hh'], pb['W_hh'],
                        pf['b_hh'].reshape(1, 3 * HID),
                        pb['b_hh'].reshape(1, 3 * HID), lengths_col)

    probs, lensplat = _scores(
        out_f, out_b, params['W1'], params['b1'].reshape(1, 64),
        params['W2'].reshape(1, 1, 64), lengths_col,
        params['temperature'].reshape(1, 1))
    att = _att_sc(probs, lensplat)
    tens, ones = _pool(out_f, out_b, att, params['Wt'], params['Wo'],
                       params['bt'].reshape(1, 11), params['bo'].reshape(1, 10))
    return tens, ones


# speedup vs baseline: 1.0243x; 1.0243x over previous
import functools

import jax
import jax.numpy as jnp
from jax import lax
from jax.experimental import pallas as pl
from jax.experimental.pallas import tpu as pltpu
from jax.experimental.pallas import tpu_sc as plsc

B, T, C, H, W = 32, 256, 3, 32, 32
FEAT = 512
HID = 128
TOP_K = 8
CHW = C * H * W

_DN_T = (((1,), (1,)), ((), ()))


def _dot_t(a, b):
    return jax.lax.dot_general(a, b, _DN_T,
                               preferred_element_type=jnp.float32)


_DN_LT = (((0,), (1,)), ((), ()))

B_BLK = 4


def _enc_kernel(x_ref, wenc_ref, benc_ref, wf_ref, wb_ref, bf_ref, bb_ref,
                out_ref):
    for j in range(B_BLK):
        feats = jax.lax.dot_general(x_ref[j], wenc_ref[...], _DN_LT,
                                    preferred_element_type=jnp.float32)
        feats = feats + benc_ref[...]
        out_ref[j, :, :3 * HID] = _dot_t(feats, wf_ref[...]) + bf_ref[...]
        out_ref[j, :, 3 * HID:] = _dot_t(feats, wb_ref[...]) + bb_ref[...]


def _encode(xt, wenc, benc, wf, wb, bf, bb):
    return pl.pallas_call(
        _enc_kernel,
        grid=(B // B_BLK,),
        in_specs=[
            pl.BlockSpec((B_BLK, CHW, T), lambda k: (k, 0, 0)),
            pl.BlockSpec((FEAT, CHW), lambda k: (0, 0)),
            pl.BlockSpec((1, FEAT), lambda k: (0, 0)),
            pl.BlockSpec((3 * HID, FEAT), lambda k: (0, 0)),
            pl.BlockSpec((3 * HID, FEAT), lambda k: (0, 0)),
            pl.BlockSpec((1, 3 * HID), lambda k: (0, 0)),
            pl.BlockSpec((1, 3 * HID), lambda k: (0, 0)),
        ],
        out_specs=pl.BlockSpec((B_BLK, T, 6 * HID), lambda k: (k, 0, 0)),
        out_shape=jax.ShapeDtypeStruct((B, T, 6 * HID), jnp.float32),
    )(xt, wenc, benc, wf, wb, bf, bb)


SEQ_BLK = 64


def _gru_kernel(gf_ref, gb_ref, whf_ref, whb_ref, bhf_ref, bhb_ref, len_ref,
                outf_ref, outb_ref, h_ref):
    k = pl.program_id(0)

    @pl.when(k == 0)
    def _():
        h_ref[...] = jnp.zeros_like(h_ref)

    lengths = len_ref[...]

    def step(i, _):
        h = h_ref[...]
        gi_f = gf_ref[:, i, :]
        gi_b = gb_ref[:, SEQ_BLK - 1 - i, :]
        gh_f = _dot_t(h[:, :HID], whf_ref[...]) + bhf_ref[...]
        gh_b = _dot_t(h[:, HID:], whb_ref[...]) + bhb_ref[...]

        t_f = k * SEQ_BLK + i
        t_b = T - 1 - t_f

        def gru_dir(gi, gh_d, h_d, t):
            r = jax.nn.sigmoid(gi[:, :HID] + gh_d[:, :HID])
            z = jax.nn.sigmoid(gi[:, HID:2 * HID] + gh_d[:, HID:2 * HID])
            n = jnp.tanh(gi[:, 2 * HID:] + r * gh_d[:, 2 * HID:])
            h_new = (1.0 - z) * n + z * h_d
            valid = lengths > t
            return jnp.where(valid, h_new, h_d)

        h_f = gru_dir(gi_f, gh_f, h[:, :HID], t_f)
        h_b = gru_dir(gi_b, gh_b, h[:, HID:], t_b)
        outf_ref[:, i, :] = h_f
        outb_ref[:, SEQ_BLK - 1 - i, :] = h_b
        h_ref[...] = jnp.concatenate([h_f, h_b], axis=1)
        return 0

    jax.lax.fori_loop(0, SEQ_BLK, step, 0, unroll=True)


def _gru(gcat3, whf, whb, bhf, bhb, lengths_col):
    nb = T // SEQ_BLK
    out = pl.pallas_call(
        _gru_kernel,
        grid=(nb,),
        in_specs=[
            pl.BlockSpec((B, SEQ_BLK, 3 * HID), lambda k: (0, k, 0)),
            pl.BlockSpec((B, SEQ_BLK, 3 * HID), lambda k, nb=nb: (0, nb - 1 - k, 1)),
            pl.BlockSpec((3 * HID, HID), lambda k: (0, 0)),
            pl.BlockSpec((3 * HID, HID), lambda k: (0, 0)),
            pl.BlockSpec((1, 3 * HID), lambda k: (0, 0)),
            pl.BlockSpec((1, 3 * HID), lambda k: (0, 0)),
            pl.BlockSpec((B, 1), lambda k: (0, 0)),
        ],
        out_specs=[
            pl.BlockSpec((B, SEQ_BLK, HID), lambda k: (0, k, 0)),
            pl.BlockSpec((B, SEQ_BLK, HID), lambda k, nb=nb: (0, nb - 1 - k, 0)),
        ],
        out_shape=[
            jax.ShapeDtypeStruct((B, T, HID), jnp.float32),
            jax.ShapeDtypeStruct((B, T, HID), jnp.float32),
        ],
        scratch_shapes=[pltpu.VMEM((B, 2 * HID), jnp.float32)],
        compiler_params=pltpu.CompilerParams(
            dimension_semantics=("arbitrary",)),
    )(gcat3, gcat3, whf, whb, bhf, bhb, lengths_col)
    return out


def _score_kernel(xf_ref, xb_ref, w1_ref, b1_ref, w2_ref, len_ref, temp_ref,
                  probs_ref, lsp_ref):
    xf = xf_ref[...]
    xb = xb_ref[...]
    w1 = w1_ref[...]
    h1 = _dot_t(xf.reshape(B * T, HID), w1[:, :HID])
    h1 = h1 + _dot_t(xb.reshape(B * T, HID), w1[:, HID:])
    h1 = jax.nn.relu(h1 + b1_ref[...])
    scores = jnp.sum(h1.reshape(B, T, 64) * w2_ref[...], axis=2)

    lengths = len_ref[...]
    tpos = jax.lax.broadcasted_iota(jnp.int32, (B, T), 1)
    mask = tpos < lengths
    temp = jnp.clip(temp_ref[0, 0], 0.001, 10.0)
    logits = jnp.where(mask, scores * (1.0 / temp), -jnp.inf)

    m = jnp.max(logits, axis=1, keepdims=True)
    e = jnp.exp(logits - m)
    probs_ref[...] = e / jnp.sum(e, axis=1, keepdims=True)
    lsp_ref[...] = jnp.broadcast_to(lengths, (B, 16))


def _scores(out_f, out_b, w1, b1, w2, lengths_col, temp):
    full = lambda s: pl.BlockSpec(s, lambda *a: tuple(0 for _ in s))
    return pl.pallas_call(
        _score_kernel,
        in_specs=[
            full((B, T, HID)),
            full((B, T, HID)),
            full((64, 2 * HID)),
            full((1, 64)),
            full((1, 1, 64)),
            full((B, 1)),
            pl.BlockSpec(memory_space=pltpu.SMEM),
        ],
        out_specs=[full((B, T)), full((B, 16))],
        out_shape=[jax.ShapeDtypeStruct((B, T), jnp.float32),
                   jax.ShapeDtypeStruct((B, 16), jnp.int32)],
    )(out_f, out_b, w1, b1, w2, lengths_col, temp)


_SC_L = 16
_NCH = T // _SC_L


def _lane_gather(v, idx):
    dn = lax.GatherDimensionNumbers(offset_dims=(),
                                    collapsed_slice_dims=(0,),
                                    start_index_map=(0,))
    return lax.gather(v, idx[:, None], dn, (1,),
                      mode=lax.GatherScatterMode.PROMISE_IN_BOUNDS)


def _splat_reduce(v, op, iota):
    for s in (1, 2, 4, 8):
        v = op(v, _lane_gather(v, iota ^ s))
    return v


def _att_sc(probs, lensplat):
    info = plsc.get_sparse_core_info()
    nc = info.num_cores

    @functools.partial(
        pl.kernel,
        mesh=plsc.VectorSubcoreMesh(core_axis_name="c", subcore_axis_name="s"),
        out_type=jax.ShapeDtypeStruct((B, T), jnp.float32),
        scratch_types=[
            pltpu.VMEM((T,), jnp.float32),
            pltpu.VMEM((T,), jnp.float32),
            pltpu.VMEM((T,), jnp.float32),
            pltpu.VMEM((_SC_L,), jnp.int32),
        ],
    )
    def att_kernel(probs_hbm, lsp_hbm, att_hbm, work_v, orig_v, out_v, len_v):
        wid = lax.axis_index("s") * nc + lax.axis_index("c")
        pltpu.sync_copy(probs_hbm.at[wid], work_v)
        pltpu.sync_copy(probs_hbm.at[wid], orig_v)
        pltpu.sync_copy(lsp_hbm.at[wid], len_v)

        iota = lax.iota(jnp.int32, _SC_L)
        lvalv = len_v[...]
        lfv = lvalv.astype(jnp.float32)

        vsumv = jnp.zeros((_SC_L,), jnp.float32)
        for _ in range(TOP_K):
            chunks = [work_v[pl.ds(c * _SC_L, _SC_L)] for c in range(_NCH)]
            m = chunks[0]
            for c in range(1, _NCH):
                m = jnp.maximum(m, chunks[c])
            gv = _splat_reduce(m, jnp.maximum, iota)
            cmin = jnp.full((_SC_L,), T, jnp.int32)
            for c in range(_NCH):
                cand = jnp.where(chunks[c] == gv, iota + c * _SC_L, T)
                cmin = jnp.minimum(cmin, cand)
            gidxv = _splat_reduce(cmin, jnp.minimum, iota)
            vsumv = vsumv + gv
            for c in range(_NCH):
                upd = jnp.where(iota + c * _SC_L == gidxv,
                                jnp.float32(-1.0), chunks[c])
                work_v[pl.ds(c * _SC_L, _SC_L)] = upd

        inv_vsumv = 1.0 / jnp.maximum(vsumv, jnp.float32(1e-12))
        uvalv = 1.0 / (lfv + jnp.float32(1e-08))
        utv = jnp.where(vsumv > jnp.float32(1e-08),
                        jnp.full((_SC_L,), 1.0, jnp.float32),
                        jnp.full((_SC_L,), 0.0, jnp.float32))
        for c in range(_NCH):
            w = work_v[pl.ds(c * _SC_L, _SC_L)]
            o = orig_v[pl.ds(c * _SC_L, _SC_L)]
            topk_att = jnp.where(w == jnp.full((_SC_L,), -1.0, jnp.float32),
                                 o * inv_vsumv, jnp.float32(0.0))
            uni_att = jnp.where(iota + c * _SC_L < lvalv, uvalv,
                                jnp.float32(0.0))
            out_v[pl.ds(c * _SC_L, _SC_L)] = (topk_att * utv
                                              + uni_att * (1.0 - utv))
        pltpu.sync_copy(out_v, att_hbm.at[wid])

    return att_kernel(probs, lensplat)


def _pool_kernel(xf_ref, xb_ref, att_ref, wt_ref, wo_ref, bt_ref, bo_ref,
                 outt_ref, outo_ref):
    att = att_ref[...]
    seq_f = jnp.sum(xf_ref[...] * att[:, :, None], axis=1)
    seq_b = jnp.sum(xb_ref[...] * att[:, :, None], axis=1)
    wt = wt_ref[...]
    wo = wo_ref[...]
    outt_ref[...] = (_dot_t(seq_f, wt[:, :HID]) + _dot_t(seq_b, wt[:, HID:])
                     + bt_ref[...])
    outo_ref[...] = (_dot_t(seq_f, wo[:, :HID]) + _dot_t(seq_b, wo[:, HID:])
                     + bo_ref[...])


def _pool(out_f, out_b, att, wt, wo, bt, bo):
    full = lambda s: pl.BlockSpec(s, lambda *a: tuple(0 for _ in s))
    return pl.pallas_call(
        _pool_kernel,
        in_specs=[
            full((B, T, HID)),
            full((B, T, HID)),
            full((B, T)),
            full((11, 2 * HID)),
            full((10, 2 * HID)),
            full((1, 11)),
            full((1, 10)),
        ],
        out_specs=[
            full((B, 11)),
            full((B, 10)),
        ],
        out_shape=[
            jax.ShapeDtypeStruct((B, 11), jnp.float32),
            jax.ShapeDtypeStruct((B, 10), jnp.float32),
        ],
    )(out_f, out_b, att, wt, wo, bt, bo)


def kernel(frames, params, lengths):
    xt = jnp.transpose(frames, (0, 2, 3, 4, 1)).reshape(B, CHW, T)
    pf, pb = params['gru_fwd'], params['gru_bwd']

    gcat3 = _encode(xt, params['W_enc'], params['b_enc'].reshape(1, FEAT),
                    pf['W_ih'], pb['W_ih'],
                    pf['b_ih'].reshape(1, 3 * HID),
                    pb['b_ih'].reshape(1, 3 * HID))

    lengths_col = lengths.reshape(B, 1)
    out_f, out_b = _gru(gcat3, pf['W_hh'], pb['W_hh'],
                        pf['b_hh'].reshape(1, 3 * HID),
                        pb['b_hh'].reshape(1, 3 * HID), lengths_col)

    probs, lensplat = _scores(
        out_f, out_b, params['W1'], params['b1'].reshape(1, 64),
        params['W2'].reshape(1, 1, 64), lengths_col,
        params['temperature'].reshape(1, 1))
    att = _att_sc(probs, lensplat)
    tens, ones = _pool(out_f, out_b, att, params['Wt'], params['Wo'],
                       params['bt'].reshape(1, 11), params['bo'].reshape(1, 10))
    return tens, ones
